# baseline (speedup 1.0000x reference)
import functools

import jax
import jax.numpy as jnp
from jax.experimental import pallas as pl
from jax.experimental.pallas import tpu as pltpu

_LANES = 128
_TOPK = 2


def _router_kernel(E, x_ref, wg_ref, wT_ref, stats_ref):
    b = pl.program_id(0)
    c = pl.program_id(1)
    x = x_ref[0]
    logits = jnp.dot(x, wg_ref[...], preferred_element_type=jnp.float32)
    lane = jax.lax.broadcasted_iota(jnp.int32, logits.shape, 1)
    valid = lane < E
    neg = jnp.float32(-1e30)
    lg = jnp.where(valid, logits, neg)
    m1 = jnp.max(lg, axis=1, keepdims=True)
    idx1 = jnp.min(jnp.where((lg == m1) & valid, lane, _LANES), axis=1,
                   keepdims=True)
    lg2 = jnp.where(lane == idx1, neg, lg)
    m2 = jnp.max(lg2, axis=1, keepdims=True)
    idx2 = jnp.min(jnp.where((lg2 == m2) & valid, lane, _LANES), axis=1,
                   keepdims=True)
    d = m2 - m1
    lse = jnp.log1p(jnp.exp(d))
    logp1 = -lse
    logp2 = d - lse
    g1 = jnp.exp(logp1)
    g2 = jnp.exp(logp2)
    wfull = (jnp.where(lane == idx1, g1, 0.0)
             + jnp.where(lane == idx2, g2, 0.0))
    wT_ref[0] = jnp.swapaxes(wfull, 0, 1)[:wT_ref.shape[1]]

    @pl.when((b == 0) & (c == 0))
    def _init():
        stats_ref[...] = jnp.zeros_like(stats_ref)

    loadrow = jnp.sum(wfull, axis=0, keepdims=True)
    ent = -(g1 * logp1 + g2 * logp2)
    entrow = jnp.sum(jnp.broadcast_to(ent, ent.shape[:1] + (_LANES,)),
                     axis=0, keepdims=True)
    stats_ref[0:1, :] += loadrow
    stats_ref[1:2, :] += entrow


def _moe_kernel(DI, N, K, CL,
                x_ref, win_ref, wct_ref, bconv_ref, wxp_ref, wdt_ref,
                bdt_ref, at_ref, dsk_ref, wout_ref, wT_ref,
                out_ref,
                cbuf, dt_s, u_s, bp_s, cp_s, ys_s, h_s):
    e = pl.program_id(0)
    b = pl.program_id(1)
    c = pl.program_id(2)

    xz = jnp.dot(x_ref[0], win_ref[0], preferred_element_type=jnp.float32)
    xi_raw = xz[:, :DI]
    z = xz[:, DI:]

    @pl.when(c == 0)
    def _zero_tail():
        cbuf[8 - (K - 1):8, :] = jnp.zeros((K - 1, DI), jnp.float32)

    cbuf[8:8 + CL, :] = xi_raw
    conv = bconv_ref[0]
    for k in range(K):
        conv = conv + cbuf[8 - (K - 1) + k:8 - (K - 1) + k + CL, :] \
            * wct_ref[0, k:k + 1, :]
    cbuf[8 - (K - 1):8, :] = xi_raw[CL - (K - 1):, :]
    xi = conv * jax.nn.sigmoid(conv)

    xdbl = jnp.dot(xi.astype(jnp.bfloat16), wxp_ref[0],
                   preferred_element_type=jnp.float32)
    dtpre = jnp.dot(xdbl[:, 0:_LANES].astype(jnp.bfloat16), wdt_ref[0],
                    preferred_element_type=jnp.float32) + bdt_ref[0]
    dt = jnp.where(dtpre > 20.0, dtpre,
                   jnp.log1p(jnp.exp(jnp.minimum(dtpre, 20.0))))
    dt_s[0:CL, :] = dt
    u_s[0:CL, :] = dt * xi
    dt_s[CL:CL + 8, :] = jnp.zeros((8, DI), jnp.float32)
    u_s[CL:CL + 8, :] = jnp.zeros((8, DI), jnp.float32)
    bp_s[0:CL, :] = xdbl[:, _LANES:_LANES + N]
    cp_s[0:CL, :] = xdbl[:, 2 * _LANES:2 * _LANES + N]

    @pl.when(c == 0)
    def _zero_h():
        h_s[...] = jnp.zeros((N, DI), jnp.float32)

    eye = (jax.lax.broadcasted_iota(jnp.int32, (N, N), 0)
           == jax.lax.broadcasted_iota(jnp.int32, (N, N), 1))
    A2 = at_ref[0]

    def body(t, h):
        dtr = dt_s[pl.ds(t, 1), :]
        dA = jnp.exp2(A2 * dtr)
        ur = u_s[pl.ds(t, 1), :]
        brow = jnp.broadcast_to(bp_s[pl.ds(t, 1), :], (N, N))
        bcol = jnp.sum(jnp.where(eye, brow, 0.0), axis=1, keepdims=True)
        crow = cp_s[pl.ds(t, 1), :]
        h = dA * h + ur * bcol
        ys_s[pl.ds(t, 1), :] = jnp.dot(
            crow.astype(jnp.bfloat16), h.astype(jnp.bfloat16),
            preferred_element_type=jnp.float32)
        return h

    h = jax.lax.fori_loop(0, CL, body, h_s[...])
    h_s[...] = h

    y = ys_s[...] + xi * dsk_ref[0]
    y = y * (z * jax.nn.sigmoid(z))

    wrow = wT_ref[0, pl.ds(e, 1), :]
    wcol = jnp.swapaxes(wrow, 0, 1)
    contrib = jnp.dot((y * wcol).astype(jnp.bfloat16), wout_ref[0],
                      preferred_element_type=jnp.float32)

    sl = pl.ds(c * CL, CL)

    @pl.when(e == 0)
    def _first():
        out_ref[pl.ds(b, 1), sl, :] = contrib[None]

    @pl.when(e > 0)
    def _accum():
        out_ref[pl.ds(b, 1), sl, :] += contrib[None]


def kernel(x, W_gate, W_in, W_conv, b_conv, W_xproj, W_dt, b_dt, A_log,
           D_skip, W_out):
    B, L, D = x.shape
    E = W_gate.shape[0]
    DI, K = W_conv.shape[1], W_conv.shape[2]
    N = A_log.shape[2]
    DT_RANK = W_dt.shape[1]
    f32 = jnp.float32
    bf16 = jnp.bfloat16

    CL = 128 if L % 128 == 0 else L
    NC = L // CL
    Epad = max(8, -(-E // 8) * 8)

    wg_pad = jnp.pad(W_gate.T, ((0, 0), (0, _LANES - E)))
    win_bf = W_in.astype(bf16)
    wct = jnp.pad(jnp.swapaxes(W_conv, 1, 2), ((0, 0), (0, 8 - K), (0, 0)))
    bconv3 = b_conv[:, None, :]
    wxp = jnp.concatenate([
        jnp.pad(W_xproj[:, :, :DT_RANK], ((0, 0), (0, 0), (0, _LANES - DT_RANK))),
        jnp.pad(W_xproj[:, :, DT_RANK:DT_RANK + N], ((0, 0), (0, 0), (0, _LANES - N))),
        jnp.pad(W_xproj[:, :, DT_RANK + N:], ((0, 0), (0, 0), (0, _LANES - N))),
    ], axis=-1).astype(bf16)
    wdt = jnp.pad(W_dt, ((0, 0), (0, _LANES - DT_RANK), (0, 0))).astype(bf16)
    bdt3 = b_dt[:, None, :]
    at = jnp.swapaxes(-jnp.exp(A_log), 1, 2) * 1.4426950408889634
    dsk3 = D_skip[:, None, :]
    wout_bf = W_out.astype(bf16)
    x_bf = x.astype(bf16)

    wT, stats = pl.pallas_call(
        functools.partial(_router_kernel, E),
        grid=(B, NC),
        in_specs=[
            pl.BlockSpec((1, CL, D), lambda b, c: (b, c, 0)),
            pl.BlockSpec((D, _LANES), lambda b, c: (0, 0)),
        ],
        out_specs=[
            pl.BlockSpec((1, Epad, CL), lambda b, c: (b, 0, c)),
            pl.BlockSpec((8, _LANES), lambda b, c: (0, 0)),
        ],
        out_shape=[
            jax.ShapeDtypeStruct((B, Epad, L), f32),
            jax.ShapeDtypeStruct((8, _LANES), f32),
        ],
    )(x, wg_pad)

    out = pl.pallas_call(
        functools.partial(_moe_kernel, DI, N, K, CL),
        grid=(E, B, NC),
        in_specs=[
            pl.BlockSpec((1, CL, D), lambda e, b, c: (b, c, 0)),
            pl.BlockSpec((1, D, 2 * DI), lambda e, b, c: (e, 0, 0)),
            pl.BlockSpec((1, 8, DI), lambda e, b, c: (e, 0, 0)),
            pl.BlockSpec((1, 1, DI), lambda e, b, c: (e, 0, 0)),
            pl.BlockSpec((1, DI, 3 * _LANES), lambda e, b, c: (e, 0, 0)),
            pl.BlockSpec((1, _LANES, DI), lambda e, b, c: (e, 0, 0)),
            pl.BlockSpec((1, 1, DI), lambda e, b, c: (e, 0, 0)),
            pl.BlockSpec((1, N, DI), lambda e, b, c: (e, 0, 0)),
            pl.BlockSpec((1, 1, DI), lambda e, b, c: (e, 0, 0)),
            pl.BlockSpec((1, DI, D), lambda e, b, c: (e, 0, 0)),
            pl.BlockSpec((1, Epad, CL), lambda e, b, c: (b, 0, c)),
        ],
        out_specs=pl.BlockSpec((B, L, D), lambda e, b, c: (0, 0, 0)),
        out_shape=jax.ShapeDtypeStruct((B, L, D), f32),
        scratch_shapes=[
            pltpu.VMEM((CL + 8, DI), f32),
            pltpu.VMEM((CL + 8, DI), f32),
            pltpu.VMEM((CL + 8, DI), f32),
            pltpu.VMEM((CL + 8, N), f32),
            pltpu.VMEM((CL + 8, N), f32),
            pltpu.VMEM((CL, DI), f32),
            pltpu.VMEM((N, DI), f32),
        ],
        compiler_params=pltpu.CompilerParams(
            dimension_semantics=("arbitrary",) * 3,
            vmem_limit_bytes=100 * 1024 * 1024,
        ),
    )(x_bf, win_bf, wct, bconv3, wxp, wdt, bdt3, at, dsk3, wout_bf, wT)

    load = stats[0, :E]
    routing_entropy = stats[1, 0] / (B * L)
    target = B * L * _TOPK / E
    ln = load / (target + 1e-8)
    mean = jnp.clip(jnp.mean(ln), 1e-8)
    std = jnp.std(ln, ddof=1)
    lb_loss = std / mean * 0.01 * E
    return out, lb_loss, load, routing_entropy


# speedup vs baseline: 1.7232x; 1.7232x over previous
import functools

import jax
import jax.numpy as jnp
from jax.experimental import pallas as pl
from jax.experimental.pallas import tpu as pltpu
from jax.experimental.pallas import tpu_sc as plsc

_LANES = 128
_TOPK = 2
_SCV = 16


def _logits_kernel(Epad, NC, x_ref, wg_ref, lt_ref):
    logits = jnp.dot(x_ref[0], wg_ref[...],
                     preferred_element_type=jnp.float32)
    lt_ref[...] = jnp.swapaxes(logits, 0, 1)[:lt_ref.shape[0]]


def _sc_router(logitsT, E):
    Epad, T = logitsT.shape
    mesh = plsc.VectorSubcoreMesh(core_axis_name="c", subcore_axis_name="s")

    def body(lt_vmem, w_vmem, d_vmem):
        for c1 in range(0, _LANES, _SCV):
            sl = pl.ds(c1, _SCV)
            m1 = jnp.full((_SCV,), -1e30, jnp.float32)
            m2 = jnp.full((_SCV,), -1e30, jnp.float32)
            i1 = jnp.zeros((_SCV,), jnp.int32)
            i2 = jnp.zeros((_SCV,), jnp.int32)
            for e in range(E):
                le = lt_vmem[e, sl]
                gt1 = le > m1
                gt2 = le > m2
                i2 = jnp.where(gt1, i1, jnp.where(gt2, e, i2))
                m2 = jnp.where(gt1, m1, jnp.where(gt2, le, m2))
                i1 = jnp.where(gt1, e, i1)
                m1 = jnp.where(gt1, le, m1)
            d = m2 - m1
            ed = jnp.exp(d)
            g1 = 1.0 / (1.0 + ed)
            g2 = ed * g1
            for e in range(E):
                w_vmem[e, sl] = (jnp.where(i1 == e, g1, 0.0)
                                 + jnp.where(i2 == e, g2, 0.0))
            d_vmem[0, sl] = d

    @pl.kernel(out_type=[jax.ShapeDtypeStruct((Epad, T), jnp.float32),
                         jax.ShapeDtypeStruct((8, T), jnp.float32)],
               mesh=mesh)
    def k(lt_hbm, w_hbm, d_hbm):
        pltpu.emit_pipeline(
            body,
            grid=(T // _LANES,),
            in_specs=[pl.BlockSpec((Epad, _LANES), lambda i: (0, i))],
            out_specs=[pl.BlockSpec((Epad, _LANES), lambda i: (0, i)),
                       pl.BlockSpec((8, _LANES), lambda i: (0, i))],
            core_axis_name=("c", "s"),
            dimension_semantics=(pltpu.PARALLEL,),
        )(lt_hbm, w_hbm, d_hbm)

    return k(logitsT)


def _stats_kernel(w_ref, d_ref, load_ref, ent_ref):
    load = jnp.sum(w_ref[...], axis=1, keepdims=True)
    load_ref[...] = jnp.broadcast_to(load, load_ref.shape)
    d = d_ref[0:1, :]
    ed = jnp.exp(d)
    lse = jnp.log1p(ed)
    g2 = ed / (1.0 + ed)
    ent = lse - g2 * d
    ent_ref[...] = jnp.broadcast_to(
        jnp.sum(ent, axis=1, keepdims=True), ent_ref.shape)


def _moe_kernel(DI, N, K, CL,
                x_ref, win_ref, wct_ref, bconv_ref, wxp_ref, wdt_ref,
                bdt_ref, at_ref, dsk_ref, wout_ref, wT_ref,
                out_ref,
                cbuf, dt_s, u_s, bp_s, cp_s, ys_s, h_s):
    e = pl.program_id(0)
    b = pl.program_id(1)
    c = pl.program_id(2)

    xz = jnp.dot(x_ref[0], win_ref[0], preferred_element_type=jnp.float32)
    xi_raw = xz[:, :DI]
    z = xz[:, DI:]

    @pl.when(c == 0)
    def _zero_tail():
        cbuf[8 - (K - 1):8, :] = jnp.zeros((K - 1, DI), jnp.float32)

    cbuf[8:8 + CL, :] = xi_raw
    conv = bconv_ref[0]
    for k in range(K):
        conv = conv + cbuf[8 - (K - 1) + k:8 - (K - 1) + k + CL, :] \
            * wct_ref[0, k:k + 1, :]
    cbuf[8 - (K - 1):8, :] = xi_raw[CL - (K - 1):, :]
    xi = conv * jax.nn.sigmoid(conv)

    xdbl = jnp.dot(xi.astype(jnp.bfloat16), wxp_ref[0],
                   preferred_element_type=jnp.float32)
    dtpre = jnp.dot(xdbl[:, 0:_LANES].astype(jnp.bfloat16), wdt_ref[0],
                    preferred_element_type=jnp.float32) + bdt_ref[0]
    dt = jnp.where(dtpre > 20.0, dtpre,
                   jnp.log1p(jnp.exp(jnp.minimum(dtpre, 20.0))))
    dt_s[0:CL, :] = dt
    u_s[0:CL, :] = dt * xi
    dt_s[CL:CL + 8, :] = jnp.zeros((8, DI), jnp.float32)
    u_s[CL:CL + 8, :] = jnp.zeros((8, DI), jnp.float32)
    bp_s[0:CL, :] = xdbl[:, _LANES:_LANES + N]
    cp_s[0:CL, :] = xdbl[:, 2 * _LANES:2 * _LANES + N]

    @pl.when(c == 0)
    def _zero_h():
        h_s[...] = jnp.zeros((N, DI), jnp.float32)

    eye = (jax.lax.broadcasted_iota(jnp.int32, (N, N), 0)
           == jax.lax.broadcasted_iota(jnp.int32, (N, N), 1))
    A2 = at_ref[0]

    def _col(ref, t):
        row = jnp.broadcast_to(ref[pl.ds(t, 1), :], (N, N))
        return jnp.sum(jnp.where(eye, row, 0.0), axis=1, keepdims=True)

    def body(t, h):
        dtr = dt_s[pl.ds(t, 1), :]
        dA = jnp.exp2(A2 * dtr)
        ur = u_s[pl.ds(t, 1), :]
        bcol = _col(bp_s, t)
        ccol = _col(cp_s, t)
        h = dA * h + ur * bcol
        ys_s[pl.ds(t, 1), :] = jnp.sum(h * ccol, axis=0, keepdims=True)
        return h

    h = jax.lax.fori_loop(0, CL, body, h_s[...])
    h_s[...] = h

    y = ys_s[...] + xi * dsk_ref[0]
    y = y * (z * jax.nn.sigmoid(z))

    wrow = wT_ref[0, pl.ds(e, 1), :]
    wcol = jnp.swapaxes(wrow, 0, 1)
    contrib = jnp.dot((y * wcol).astype(jnp.bfloat16), wout_ref[0],
                      preferred_element_type=jnp.float32)

    sl = pl.ds(c * CL, CL)

    @pl.when(e == 0)
    def _first():
        out_ref[pl.ds(b, 1), sl, :] = contrib[None]

    @pl.when(e > 0)
    def _accum():
        out_ref[pl.ds(b, 1), sl, :] += contrib[None]


def kernel(x, W_gate, W_in, W_conv, b_conv, W_xproj, W_dt, b_dt, A_log,
           D_skip, W_out):
    B, L, D = x.shape
    E = W_gate.shape[0]
    DI, K = W_conv.shape[1], W_conv.shape[2]
    N = A_log.shape[2]
    DT_RANK = W_dt.shape[1]
    f32 = jnp.float32
    bf16 = jnp.bfloat16

    CL = 128 if L % 128 == 0 else L
    NC = L // CL
    Epad = max(8, -(-E // 8) * 8)

    wg_pad = jnp.pad(W_gate.T, ((0, 0), (0, _LANES - E)))
    win_bf = W_in.astype(bf16)
    wct = jnp.pad(jnp.swapaxes(W_conv, 1, 2), ((0, 0), (0, 8 - K), (0, 0)))
    bconv3 = b_conv[:, None, :]
    wxp = jnp.concatenate([
        jnp.pad(W_xproj[:, :, :DT_RANK], ((0, 0), (0, 0), (0, _LANES - DT_RANK))),
        jnp.pad(W_xproj[:, :, DT_RANK:DT_RANK + N], ((0, 0), (0, 0), (0, _LANES - N))),
        jnp.pad(W_xproj[:, :, DT_RANK + N:], ((0, 0), (0, 0), (0, _LANES - N))),
    ], axis=-1).astype(bf16)
    wdt = jnp.pad(W_dt, ((0, 0), (0, _LANES - DT_RANK), (0, 0))).astype(bf16)
    bdt3 = b_dt[:, None, :]
    at = jnp.swapaxes(-jnp.exp(A_log), 1, 2) * 1.4426950408889634
    dsk3 = D_skip[:, None, :]
    wout_bf = W_out.astype(bf16)
    x_bf = x.astype(bf16)

    logitsT = pl.pallas_call(
        functools.partial(_logits_kernel, Epad, NC),
        grid=(B, NC),
        in_specs=[
            pl.BlockSpec((1, CL, D), lambda b, c: (b, c, 0)),
            pl.BlockSpec((D, _LANES), lambda b, c: (0, 0)),
        ],
        out_specs=pl.BlockSpec((Epad, CL), lambda b, c, NC=NC: (0, b * NC + c)),
        out_shape=jax.ShapeDtypeStruct((Epad, B * L), f32),
    )(x, wg_pad)
    wsc, dsc = _sc_router(logitsT, E)
    loadb, entb = pl.pallas_call(
        _stats_kernel,
        grid=(1,),
        in_specs=[
            pl.BlockSpec((Epad, B * L), lambda i: (0, 0)),
            pl.BlockSpec((8, B * L), lambda i: (0, 0)),
        ],
        out_specs=[
            pl.BlockSpec((8, _LANES), lambda i: (0, 0)),
            pl.BlockSpec((8, _LANES), lambda i: (0, 0)),
        ],
        out_shape=[
            jax.ShapeDtypeStruct((8, _LANES), f32),
            jax.ShapeDtypeStruct((8, _LANES), f32),
        ],
    )(wsc, dsc)
    wT = wsc.reshape(Epad, B, L).transpose(1, 0, 2)

    out = pl.pallas_call(
        functools.partial(_moe_kernel, DI, N, K, CL),
        grid=(E, B, NC),
        in_specs=[
            pl.BlockSpec((1, CL, D), lambda e, b, c: (b, c, 0)),
            pl.BlockSpec((1, D, 2 * DI), lambda e, b, c: (e, 0, 0)),
            pl.BlockSpec((1, 8, DI), lambda e, b, c: (e, 0, 0)),
            pl.BlockSpec((1, 1, DI), lambda e, b, c: (e, 0, 0)),
            pl.BlockSpec((1, DI, 3 * _LANES), lambda e, b, c: (e, 0, 0)),
            pl.BlockSpec((1, _LANES, DI), lambda e, b, c: (e, 0, 0)),
            pl.BlockSpec((1, 1, DI), lambda e, b, c: (e, 0, 0)),
            pl.BlockSpec((1, N, DI), lambda e, b, c: (e, 0, 0)),
            pl.BlockSpec((1, 1, DI), lambda e, b, c: (e, 0, 0)),
            pl.BlockSpec((1, DI, D), lambda e, b, c: (e, 0, 0)),
            pl.BlockSpec((1, Epad, CL), lambda e, b, c: (b, 0, c)),
        ],
        out_specs=pl.BlockSpec((B, L, D), lambda e, b, c: (0, 0, 0)),
        out_shape=jax.ShapeDtypeStruct((B, L, D), f32),
        scratch_shapes=[
            pltpu.VMEM((CL + 8, DI), f32),
            pltpu.VMEM((CL + 8, DI), f32),
            pltpu.VMEM((CL + 8, DI), f32),
            pltpu.VMEM((CL + 8, N), f32),
            pltpu.VMEM((CL + 8, N), f32),
            pltpu.VMEM((CL, DI), f32),
            pltpu.VMEM((N, DI), f32),
        ],
        compiler_params=pltpu.CompilerParams(
            dimension_semantics=("arbitrary",) * 3,
            vmem_limit_bytes=100 * 1024 * 1024,
        ),
    )(x_bf, win_bf, wct, bconv3, wxp, wdt, bdt3, at, dsk3, wout_bf, wT)

    load = loadb[:E, 0]
    routing_entropy = entb[0, 0] / (B * L)
    target = B * L * _TOPK / E
    ln = load / (target + 1e-8)
    mean = jnp.clip(jnp.mean(ln), 1e-8)
    std = jnp.std(ln, ddof=1)
    lb_loss = std / mean * 0.01 * E
    return out, lb_loss, load, routing_entropy


# speedup vs baseline: 1.7780x; 1.0318x over previous
import functools

import jax
import jax.numpy as jnp
from jax.experimental import pallas as pl
from jax.experimental.pallas import tpu as pltpu
from jax.experimental.pallas import tpu_sc as plsc

_LANES = 128
_TOPK = 2
_SCV = 16


def _logits_kernel(Epad, NC, x_ref, wg_ref, lt_ref):
    logits = jnp.dot(x_ref[0], wg_ref[...],
                     preferred_element_type=jnp.float32)
    lt_ref[...] = jnp.swapaxes(logits, 0, 1)[:lt_ref.shape[0]]


def _sc_router(logitsT, E):
    Epad, T = logitsT.shape
    mesh = plsc.VectorSubcoreMesh(core_axis_name="c", subcore_axis_name="s")

    def body(lt_vmem, w_vmem, d_vmem):
        for c1 in range(0, _LANES, _SCV):
            sl = pl.ds(c1, _SCV)
            m1 = jnp.full((_SCV,), -1e30, jnp.float32)
            m2 = jnp.full((_SCV,), -1e30, jnp.float32)
            i1 = jnp.zeros((_SCV,), jnp.int32)
            i2 = jnp.zeros((_SCV,), jnp.int32)
            for e in range(E):
                le = lt_vmem[e, sl]
                gt1 = le > m1
                gt2 = le > m2
                i2 = jnp.where(gt1, i1, jnp.where(gt2, e, i2))
                m2 = jnp.where(gt1, m1, jnp.where(gt2, le, m2))
                i1 = jnp.where(gt1, e, i1)
                m1 = jnp.where(gt1, le, m1)
            d = m2 - m1
            ed = jnp.exp(d)
            g1 = 1.0 / (1.0 + ed)
            g2 = ed * g1
            for e in range(E):
                w_vmem[e, sl] = (jnp.where(i1 == e, g1, 0.0)
                                 + jnp.where(i2 == e, g2, 0.0))
            d_vmem[0, sl] = d

    @pl.kernel(out_type=[jax.ShapeDtypeStruct((Epad, T), jnp.float32),
                         jax.ShapeDtypeStruct((8, T), jnp.float32)],
               mesh=mesh)
    def k(lt_hbm, w_hbm, d_hbm):
        pltpu.emit_pipeline(
            body,
            grid=(T // _LANES,),
            in_specs=[pl.BlockSpec((Epad, _LANES), lambda i: (0, i))],
            out_specs=[pl.BlockSpec((Epad, _LANES), lambda i: (0, i)),
                       pl.BlockSpec((8, _LANES), lambda i: (0, i))],
            core_axis_name=("c", "s"),
            dimension_semantics=(pltpu.PARALLEL,),
        )(lt_hbm, w_hbm, d_hbm)

    return k(logitsT)


def _stats_kernel(w_ref, d_ref, load_ref, ent_ref):
    load = jnp.sum(w_ref[...], axis=1, keepdims=True)
    load_ref[...] = jnp.broadcast_to(load, load_ref.shape)
    d = d_ref[0:1, :]
    ed = jnp.exp(d)
    lse = jnp.log1p(ed)
    g2 = ed / (1.0 + ed)
    ent = lse - g2 * d
    ent_ref[...] = jnp.broadcast_to(
        jnp.sum(ent, axis=1, keepdims=True), ent_ref.shape)


def _moe_kernel(DI, N, K, CL,
                x_ref, win_ref, wct_ref, bconv_ref, wxp_ref, wdt_ref,
                bdt_ref, at_ref, dsk_ref, wout_ref, wT_ref,
                out_ref,
                cbuf, dt_s, u_s, bp_s, cp_s, ys_s, h_s):
    e = pl.program_id(0)
    b = pl.program_id(1)
    c = pl.program_id(2)

    xz = jnp.dot(x_ref[0], win_ref[0], preferred_element_type=jnp.float32)
    xi_raw = xz[:, :DI]
    z = xz[:, DI:]

    @pl.when(c == 0)
    def _zero_tail():
        cbuf[8 - (K - 1):8, :] = jnp.zeros((K - 1, DI), jnp.float32)

    cbuf[8:8 + CL, :] = xi_raw
    conv = bconv_ref[0]
    for k in range(K):
        conv = conv + cbuf[8 - (K - 1) + k:8 - (K - 1) + k + CL, :] \
            * wct_ref[0, k:k + 1, :]
    cbuf[8 - (K - 1):8, :] = xi_raw[CL - (K - 1):, :]
    xi = conv * jax.nn.sigmoid(conv)

    xdbl = jnp.dot(xi.astype(jnp.bfloat16), wxp_ref[0],
                   preferred_element_type=jnp.float32)
    dtpre = jnp.dot(xdbl[:, 0:_LANES].astype(jnp.bfloat16), wdt_ref[0],
                    preferred_element_type=jnp.float32) + bdt_ref[0]
    dt = jnp.where(dtpre > 20.0, dtpre,
                   jnp.log1p(jnp.exp(jnp.minimum(dtpre, 20.0))))
    dt_s[0:CL, :] = dt
    u_s[0:CL, :] = dt * xi
    dt_s[CL:CL + 8, :] = jnp.zeros((8, DI), jnp.float32)
    u_s[CL:CL + 8, :] = jnp.zeros((8, DI), jnp.float32)
    bp_s[0:CL, :] = xdbl[:, _LANES:_LANES + N]
    cp_s[0:CL, :] = xdbl[:, 2 * _LANES:2 * _LANES + N]

    @pl.when(c == 0)
    def _zero_h():
        h_s[...] = jnp.zeros((N, DI), jnp.float32)

    eye = (jax.lax.broadcasted_iota(jnp.int32, (N, N), 0)
           == jax.lax.broadcasted_iota(jnp.int32, (N, N), 1))
    A2 = at_ref[0]

    def _col(ref, t):
        row = jnp.broadcast_to(ref[pl.ds(t, 1), :], (N, N))
        return jnp.sum(jnp.where(eye, row, 0.0), axis=1, keepdims=True)

    def body(t, h):
        dtr = dt_s[pl.ds(t, 1), :]
        dA = jnp.exp2(A2 * dtr)
        ur = u_s[pl.ds(t, 1), :]
        bcol = _col(bp_s, t)
        ccol = _col(cp_s, t)
        h = dA * h + ur * bcol
        ys_s[pl.ds(t, 1), :] = jnp.sum(h * ccol, axis=0, keepdims=True)
        return h

    h = jax.lax.fori_loop(0, CL, body, h_s[...])
    h_s[...] = h

    y = ys_s[...] + xi * dsk_ref[0]
    y = y * (z * jax.nn.sigmoid(z))

    wrow = wT_ref[0, pl.ds(e, 1), :]
    wcol = jnp.swapaxes(wrow, 0, 1)
    contrib = jnp.dot((y * wcol).astype(jnp.bfloat16), wout_ref[0],
                      preferred_element_type=jnp.float32)

    sl = pl.ds(c * CL, CL)

    @pl.when(e == 0)
    def _first():
        out_ref[pl.ds(b, 1), sl, :] = contrib[None]

    @pl.when(e > 0)
    def _accum():
        out_ref[pl.ds(b, 1), sl, :] += contrib[None]


def kernel(x, W_gate, W_in, W_conv, b_conv, W_xproj, W_dt, b_dt, A_log,
           D_skip, W_out):
    B, L, D = x.shape
    E = W_gate.shape[0]
    DI, K = W_conv.shape[1], W_conv.shape[2]
    N = A_log.shape[2]
    DT_RANK = W_dt.shape[1]
    f32 = jnp.float32
    bf16 = jnp.bfloat16

    CL = 256 if L % 256 == 0 else (128 if L % 128 == 0 else L)
    NC = L // CL
    Epad = max(8, -(-E // 8) * 8)

    wg_pad = jnp.pad(W_gate.T, ((0, 0), (0, _LANES - E)))
    win_bf = W_in.astype(bf16)
    wct = jnp.pad(jnp.swapaxes(W_conv, 1, 2), ((0, 0), (0, 8 - K), (0, 0)))
    bconv3 = b_conv[:, None, :]
    wxp = jnp.concatenate([
        jnp.pad(W_xproj[:, :, :DT_RANK], ((0, 0), (0, 0), (0, _LANES - DT_RANK))),
        jnp.pad(W_xproj[:, :, DT_RANK:DT_RANK + N], ((0, 0), (0, 0), (0, _LANES - N))),
        jnp.pad(W_xproj[:, :, DT_RANK + N:], ((0, 0), (0, 0), (0, _LANES - N))),
    ], axis=-1).astype(bf16)
    wdt = jnp.pad(W_dt, ((0, 0), (0, _LANES - DT_RANK), (0, 0))).astype(bf16)
    bdt3 = b_dt[:, None, :]
    at = jnp.swapaxes(-jnp.exp(A_log), 1, 2) * 1.4426950408889634
    dsk3 = D_skip[:, None, :]
    wout_bf = W_out.astype(bf16)
    x_bf = x.astype(bf16)

    logitsT = pl.pallas_call(
        functools.partial(_logits_kernel, Epad, NC),
        grid=(B, NC),
        in_specs=[
            pl.BlockSpec((1, CL, D), lambda b, c: (b, c, 0)),
            pl.BlockSpec((D, _LANES), lambda b, c: (0, 0)),
        ],
        out_specs=pl.BlockSpec((Epad, CL), lambda b, c, NC=NC: (0, b * NC + c)),
        out_shape=jax.ShapeDtypeStruct((Epad, B * L), f32),
    )(x, wg_pad)
    wsc, dsc = _sc_router(logitsT, E)
    loadb, entb = pl.pallas_call(
        _stats_kernel,
        grid=(1,),
        in_specs=[
            pl.BlockSpec((Epad, B * L), lambda i: (0, 0)),
            pl.BlockSpec((8, B * L), lambda i: (0, 0)),
        ],
        out_specs=[
            pl.BlockSpec((8, _LANES), lambda i: (0, 0)),
            pl.BlockSpec((8, _LANES), lambda i: (0, 0)),
        ],
        out_shape=[
            jax.ShapeDtypeStruct((8, _LANES), f32),
            jax.ShapeDtypeStruct((8, _LANES), f32),
        ],
    )(wsc, dsc)
    wT = wsc.reshape(Epad, B, L).transpose(1, 0, 2)

    out = pl.pallas_call(
        functools.partial(_moe_kernel, DI, N, K, CL),
        grid=(E, B, NC),
        in_specs=[
            pl.BlockSpec((1, CL, D), lambda e, b, c: (b, c, 0)),
            pl.BlockSpec((1, D, 2 * DI), lambda e, b, c: (e, 0, 0)),
            pl.BlockSpec((1, 8, DI), lambda e, b, c: (e, 0, 0)),
            pl.BlockSpec((1, 1, DI), lambda e, b, c: (e, 0, 0)),
            pl.BlockSpec((1, DI, 3 * _LANES), lambda e, b, c: (e, 0, 0)),
            pl.BlockSpec((1, _LANES, DI), lambda e, b, c: (e, 0, 0)),
            pl.BlockSpec((1, 1, DI), lambda e, b, c: (e, 0, 0)),
            pl.BlockSpec((1, N, DI), lambda e, b, c: (e, 0, 0)),
            pl.BlockSpec((1, 1, DI), lambda e, b, c: (e, 0, 0)),
            pl.BlockSpec((1, DI, D), lambda e, b, c: (e, 0, 0)),
            pl.BlockSpec((1, Epad, CL), lambda e, b, c: (b, 0, c)),
        ],
        out_specs=pl.BlockSpec((B, L, D), lambda e, b, c: (0, 0, 0)),
        out_shape=jax.ShapeDtypeStruct((B, L, D), f32),
        scratch_shapes=[
            pltpu.VMEM((CL + 8, DI), f32),
            pltpu.VMEM((CL + 8, DI), f32),
            pltpu.VMEM((CL + 8, DI), f32),
            pltpu.VMEM((CL + 8, N), f32),
            pltpu.VMEM((CL + 8, N), f32),
            pltpu.VMEM((CL, DI), f32),
            pltpu.VMEM((N, DI), f32),
        ],
        compiler_params=pltpu.CompilerParams(
            dimension_semantics=("arbitrary",) * 3,
            vmem_limit_bytes=100 * 1024 * 1024,
        ),
    )(x_bf, win_bf, wct, bconv3, wxp, wdt, bdt3, at, dsk3, wout_bf, wT)

    load = loadb[:E, 0]
    routing_entropy = entb[0, 0] / (B * L)
    target = B * L * _TOPK / E
    ln = load / (target + 1e-8)
    mean = jnp.clip(jnp.mean(ln), 1e-8)
    std = jnp.std(ln, ddof=1)
    lb_loss = std / mean * 0.01 * E
    return out, lb_loss, load, routing_entropy


# speedup vs baseline: 2.3430x; 1.3178x over previous
import functools

import jax
import jax.numpy as jnp
from jax.experimental import pallas as pl
from jax.experimental.pallas import tpu as pltpu
from jax.experimental.pallas import tpu_sc as plsc

_LANES = 128
_TOPK = 2
_SCV = 16


def _logits_kernel(Epad, NC, x_ref, wg_ref, lt_ref):
    logits = jnp.dot(x_ref[0], wg_ref[...],
                     preferred_element_type=jnp.float32)
    lt_ref[...] = jnp.swapaxes(logits, 0, 1)[:lt_ref.shape[0]]


def _sc_router(logitsT, E):
    Epad, T = logitsT.shape
    mesh = plsc.VectorSubcoreMesh(core_axis_name="c", subcore_axis_name="s")

    def body(lt_vmem, w_vmem, d_vmem):
        for c1 in range(0, _LANES, _SCV):
            sl = pl.ds(c1, _SCV)
            m1 = jnp.full((_SCV,), -1e30, jnp.float32)
            m2 = jnp.full((_SCV,), -1e30, jnp.float32)
            i1 = jnp.zeros((_SCV,), jnp.int32)
            i2 = jnp.zeros((_SCV,), jnp.int32)
            for e in range(E):
                le = lt_vmem[e, sl]
                gt1 = le > m1
                gt2 = le > m2
                i2 = jnp.where(gt1, i1, jnp.where(gt2, e, i2))
                m2 = jnp.where(gt1, m1, jnp.where(gt2, le, m2))
                i1 = jnp.where(gt1, e, i1)
                m1 = jnp.where(gt1, le, m1)
            d = m2 - m1
            ed = jnp.exp(d)
            g1 = 1.0 / (1.0 + ed)
            g2 = ed * g1
            for e in range(E):
                w_vmem[e, sl] = (jnp.where(i1 == e, g1, 0.0)
                                 + jnp.where(i2 == e, g2, 0.0))
            d_vmem[0, sl] = d

    @pl.kernel(out_type=[jax.ShapeDtypeStruct((Epad, T), jnp.float32),
                         jax.ShapeDtypeStruct((8, T), jnp.float32)],
               mesh=mesh)
    def k(lt_hbm, w_hbm, d_hbm):
        pltpu.emit_pipeline(
            body,
            grid=(T // _LANES,),
            in_specs=[pl.BlockSpec((Epad, _LANES), lambda i: (0, i))],
            out_specs=[pl.BlockSpec((Epad, _LANES), lambda i: (0, i)),
                       pl.BlockSpec((8, _LANES), lambda i: (0, i))],
            core_axis_name=("c", "s"),
            dimension_semantics=(pltpu.PARALLEL,),
        )(lt_hbm, w_hbm, d_hbm)

    return k(logitsT)


def _stats_kernel(w_ref, d_ref, load_ref, ent_ref):
    load = jnp.sum(w_ref[...], axis=1, keepdims=True)
    load_ref[...] = jnp.broadcast_to(load, load_ref.shape)
    d = d_ref[0:1, :]
    ed = jnp.exp(d)
    lse = jnp.log1p(ed)
    g2 = ed / (1.0 + ed)
    ent = lse - g2 * d
    ent_ref[...] = jnp.broadcast_to(
        jnp.sum(ent, axis=1, keepdims=True), ent_ref.shape)


def _moe_kernel(DI, N, K, CL,
                x_ref, win_ref, wct_ref, bconv_ref, wxp_ref, wdt_ref,
                bdt_ref, at_ref, dsk_ref, wout_ref, wT_ref,
                out_ref,
                cbuf, dt_s, u_s, bp_s, cp_s, ys_s, h_s):
    e = pl.program_id(0)
    b = pl.program_id(1)
    c = pl.program_id(2)

    xz = jnp.dot(x_ref[0], win_ref[0], preferred_element_type=jnp.float32)
    xi_raw = xz[:, :DI]
    z = xz[:, DI:]

    @pl.when(c == 0)
    def _zero_tail():
        cbuf[8 - (K - 1):8, :] = jnp.zeros((K - 1, DI), jnp.float32)

    cbuf[8:8 + CL, :] = xi_raw
    conv = bconv_ref[0]
    for k in range(K):
        conv = conv + cbuf[8 - (K - 1) + k:8 - (K - 1) + k + CL, :] \
            * wct_ref[0, k:k + 1, :]
    cbuf[8 - (K - 1):8, :] = xi_raw[CL - (K - 1):, :]
    xi = conv * jax.nn.sigmoid(conv)

    xdbl = jnp.dot(xi.astype(jnp.bfloat16), wxp_ref[0],
                   preferred_element_type=jnp.float32)
    dtpre = jnp.dot(xdbl[:, 0:_LANES].astype(jnp.bfloat16), wdt_ref[0],
                    preferred_element_type=jnp.float32) + bdt_ref[0]
    dt = jnp.where(dtpre > 20.0, dtpre,
                   jnp.log1p(jnp.exp(jnp.minimum(dtpre, 20.0))))
    dt_s[0:CL, :] = dt
    u_s[0:CL, :] = dt * xi
    dt_s[CL:CL + 8, :] = jnp.zeros((8, DI), jnp.float32)
    u_s[CL:CL + 8, :] = jnp.zeros((8, DI), jnp.float32)
    bp_s[0:CL, :] = xdbl[:, _LANES:_LANES + N]
    cp_s[0:CL, :] = xdbl[:, 2 * _LANES:2 * _LANES + N]

    @pl.when(c == 0)
    def _zero_h():
        h_s[...] = jnp.zeros((N, DI), jnp.float32)

    eye = (jax.lax.broadcasted_iota(jnp.int32, (N, N), 0)
           == jax.lax.broadcasted_iota(jnp.int32, (N, N), 1))
    A2 = at_ref[0]

    def _col(ref, t):
        row = jnp.broadcast_to(ref[pl.ds(t, 1), :], (N, N))
        return jnp.sum(jnp.where(eye, row, 0.0), axis=1, keepdims=True)

    def body(t, h):
        dtr = dt_s[pl.ds(t, 1), :]
        dA = jnp.exp2(A2 * dtr)
        ur = u_s[pl.ds(t, 1), :]
        bcol = _col(bp_s, t)
        ccol = _col(cp_s, t)
        h = dA * h + ur * bcol
        ys_s[pl.ds(t, 1), :] = jnp.sum(h * ccol, axis=0, keepdims=True)
        return h

    def body2(i, h):
        h = body(2 * i, h)
        return body(2 * i + 1, h)

    h = jax.lax.fori_loop(0, CL // 2, body2, h_s[...])
    h_s[...] = h

    y = ys_s[...] + xi * dsk_ref[0]
    y = y * (z * jax.nn.sigmoid(z))

    wrow = wT_ref[0, pl.ds(e, 1), :]
    wcol = jnp.swapaxes(wrow, 0, 1)
    contrib = jnp.dot((y * wcol).astype(jnp.bfloat16), wout_ref[0],
                      preferred_element_type=jnp.float32)

    sl = pl.ds(c * CL, CL)

    @pl.when(e == 0)
    def _first():
        out_ref[pl.ds(b, 1), sl, :] = contrib[None]

    @pl.when(e > 0)
    def _accum():
        out_ref[pl.ds(b, 1), sl, :] += contrib[None]


def kernel(x, W_gate, W_in, W_conv, b_conv, W_xproj, W_dt, b_dt, A_log,
           D_skip, W_out):
    B, L, D = x.shape
    E = W_gate.shape[0]
    DI, K = W_conv.shape[1], W_conv.shape[2]
    N = A_log.shape[2]
    DT_RANK = W_dt.shape[1]
    f32 = jnp.float32
    bf16 = jnp.bfloat16

    CL = 256 if L % 256 == 0 else (128 if L % 128 == 0 else L)
    NC = L // CL
    Epad = max(8, -(-E // 8) * 8)

    wg_pad = jnp.pad(W_gate.T, ((0, 0), (0, _LANES - E)))
    win_bf = W_in.astype(bf16)
    wct = jnp.pad(jnp.swapaxes(W_conv, 1, 2), ((0, 0), (0, 8 - K), (0, 0)))
    bconv3 = b_conv[:, None, :]
    wxp = jnp.concatenate([
        jnp.pad(W_xproj[:, :, :DT_RANK], ((0, 0), (0, 0), (0, _LANES - DT_RANK))),
        jnp.pad(W_xproj[:, :, DT_RANK:DT_RANK + N], ((0, 0), (0, 0), (0, _LANES - N))),
        jnp.pad(W_xproj[:, :, DT_RANK + N:], ((0, 0), (0, 0), (0, _LANES - N))),
    ], axis=-1).astype(bf16)
    wdt = jnp.pad(W_dt, ((0, 0), (0, _LANES - DT_RANK), (0, 0))).astype(bf16)
    bdt3 = b_dt[:, None, :]
    at = jnp.swapaxes(-jnp.exp(A_log), 1, 2) * 1.4426950408889634
    dsk3 = D_skip[:, None, :]
    wout_bf = W_out.astype(bf16)
    x_bf = x.astype(bf16)

    logitsT = pl.pallas_call(
        functools.partial(_logits_kernel, Epad, NC),
        grid=(B, NC),
        in_specs=[
            pl.BlockSpec((1, CL, D), lambda b, c: (b, c, 0)),
            pl.BlockSpec((D, _LANES), lambda b, c: (0, 0)),
        ],
        out_specs=pl.BlockSpec((Epad, CL), lambda b, c, NC=NC: (0, b * NC + c)),
        out_shape=jax.ShapeDtypeStruct((Epad, B * L), f32),
    )(x, wg_pad)
    wsc, dsc = _sc_router(logitsT, E)
    loadb, entb = pl.pallas_call(
        _stats_kernel,
        grid=(1,),
        in_specs=[
            pl.BlockSpec((Epad, B * L), lambda i: (0, 0)),
            pl.BlockSpec((8, B * L), lambda i: (0, 0)),
        ],
        out_specs=[
            pl.BlockSpec((8, _LANES), lambda i: (0, 0)),
            pl.BlockSpec((8, _LANES), lambda i: (0, 0)),
        ],
        out_shape=[
            jax.ShapeDtypeStruct((8, _LANES), f32),
            jax.ShapeDtypeStruct((8, _LANES), f32),
        ],
    )(wsc, dsc)
    wT = wsc.reshape(Epad, B, L).transpose(1, 0, 2)

    out = pl.pallas_call(
        functools.partial(_moe_kernel, DI, N, K, CL),
        grid=(E, B, NC),
        in_specs=[
            pl.BlockSpec((1, CL, D), lambda e, b, c: (b, c, 0)),
            pl.BlockSpec((1, D, 2 * DI), lambda e, b, c: (e, 0, 0)),
            pl.BlockSpec((1, 8, DI), lambda e, b, c: (e, 0, 0)),
            pl.BlockSpec((1, 1, DI), lambda e, b, c: (e, 0, 0)),
            pl.BlockSpec((1, DI, 3 * _LANES), lambda e, b, c: (e, 0, 0)),
            pl.BlockSpec((1, _LANES, DI), lambda e, b, c: (e, 0, 0)),
            pl.BlockSpec((1, 1, DI), lambda e, b, c: (e, 0, 0)),
            pl.BlockSpec((1, N, DI), lambda e, b, c: (e, 0, 0)),
            pl.BlockSpec((1, 1, DI), lambda e, b, c: (e, 0, 0)),
            pl.BlockSpec((1, DI, D), lambda e, b, c: (e, 0, 0)),
            pl.BlockSpec((1, Epad, CL), lambda e, b, c: (b, 0, c)),
        ],
        out_specs=pl.BlockSpec((B, L, D), lambda e, b, c: (0, 0, 0)),
        out_shape=jax.ShapeDtypeStruct((B, L, D), f32),
        scratch_shapes=[
            pltpu.VMEM((CL + 8, DI), f32),
            pltpu.VMEM((CL + 8, DI), f32),
            pltpu.VMEM((CL + 8, DI), f32),
            pltpu.VMEM((CL + 8, N), f32),
            pltpu.VMEM((CL + 8, N), f32),
            pltpu.VMEM((CL, DI), f32),
            pltpu.VMEM((N, DI), f32),
        ],
        compiler_params=pltpu.CompilerParams(
            dimension_semantics=("arbitrary",) * 3,
            vmem_limit_bytes=100 * 1024 * 1024,
        ),
    )(x_bf, win_bf, wct, bconv3, wxp, wdt, bdt3, at, dsk3, wout_bf, wT)

    load = loadb[:E, 0]
    routing_entropy = entb[0, 0] / (B * L)
    target = B * L * _TOPK / E
    ln = load / (target + 1e-8)
    mean = jnp.clip(jnp.mean(ln), 1e-8)
    std = jnp.std(ln, ddof=1)
    lb_loss = std / mean * 0.01 * E
    return out, lb_loss, load, routing_entropy


# speedup vs baseline: 2.6316x; 1.1232x over previous
import functools

import jax
import jax.numpy as jnp
from jax.experimental import pallas as pl
from jax.experimental.pallas import tpu as pltpu
from jax.experimental.pallas import tpu_sc as plsc

_LANES = 128
_TOPK = 2
_SCV = 16


def _logits_kernel(Epad, NC, x_ref, wg_ref, lt_ref):
    logits = jnp.dot(x_ref[0], wg_ref[...],
                     preferred_element_type=jnp.float32)
    lt_ref[...] = jnp.swapaxes(logits, 0, 1)[:lt_ref.shape[0]]


def _sc_router(logitsT, E):
    Epad, T = logitsT.shape
    mesh = plsc.VectorSubcoreMesh(core_axis_name="c", subcore_axis_name="s")

    def body(lt_vmem, w_vmem, d_vmem):
        for c1 in range(0, _LANES, _SCV):
            sl = pl.ds(c1, _SCV)
            m1 = jnp.full((_SCV,), -1e30, jnp.float32)
            m2 = jnp.full((_SCV,), -1e30, jnp.float32)
            i1 = jnp.zeros((_SCV,), jnp.int32)
            i2 = jnp.zeros((_SCV,), jnp.int32)
            for e in range(E):
                le = lt_vmem[e, sl]
                gt1 = le > m1
                gt2 = le > m2
                i2 = jnp.where(gt1, i1, jnp.where(gt2, e, i2))
                m2 = jnp.where(gt1, m1, jnp.where(gt2, le, m2))
                i1 = jnp.where(gt1, e, i1)
                m1 = jnp.where(gt1, le, m1)
            d = m2 - m1
            ed = jnp.exp(d)
            g1 = 1.0 / (1.0 + ed)
            g2 = ed * g1
            for e in range(E):
                w_vmem[e, sl] = (jnp.where(i1 == e, g1, 0.0)
                                 + jnp.where(i2 == e, g2, 0.0))
            d_vmem[0, sl] = d

    @pl.kernel(out_type=[jax.ShapeDtypeStruct((Epad, T), jnp.float32),
                         jax.ShapeDtypeStruct((8, T), jnp.float32)],
               mesh=mesh)
    def k(lt_hbm, w_hbm, d_hbm):
        pltpu.emit_pipeline(
            body,
            grid=(T // _LANES,),
            in_specs=[pl.BlockSpec((Epad, _LANES), lambda i: (0, i))],
            out_specs=[pl.BlockSpec((Epad, _LANES), lambda i: (0, i)),
                       pl.BlockSpec((8, _LANES), lambda i: (0, i))],
            core_axis_name=("c", "s"),
            dimension_semantics=(pltpu.PARALLEL,),
        )(lt_hbm, w_hbm, d_hbm)

    return k(logitsT)


def _stats_kernel(w_ref, d_ref, load_ref, ent_ref):
    load = jnp.sum(w_ref[...], axis=1, keepdims=True)
    load_ref[...] = jnp.broadcast_to(load, load_ref.shape)
    d = d_ref[0:1, :]
    ed = jnp.exp(d)
    lse = jnp.log1p(ed)
    g2 = ed / (1.0 + ed)
    ent = lse - g2 * d
    ent_ref[...] = jnp.broadcast_to(
        jnp.sum(ent, axis=1, keepdims=True), ent_ref.shape)


def _moe_kernel(DI, N, K, CL,
                x_ref, win_ref, wct_ref, bconv_ref, wxp_ref, wdt_ref,
                bdt_ref, at_ref, dsk_ref, wout_ref, wT_ref,
                out_ref,
                cbuf, dt_s, u_s, bp_s, cp_s, ys_s, h_s):
    e = pl.program_id(0)
    b = pl.program_id(1)
    c = pl.program_id(2)

    xz = jnp.dot(x_ref[0], win_ref[0], preferred_element_type=jnp.float32)
    xi_raw = xz[:, :DI]
    z = xz[:, DI:]

    @pl.when(c == 0)
    def _zero_tail():
        cbuf[8 - (K - 1):8, :] = jnp.zeros((K - 1, DI), jnp.float32)

    cbuf[8:8 + CL, :] = xi_raw
    conv = bconv_ref[0]
    for k in range(K):
        conv = conv + cbuf[8 - (K - 1) + k:8 - (K - 1) + k + CL, :] \
            * wct_ref[0, k:k + 1, :]
    cbuf[8 - (K - 1):8, :] = xi_raw[CL - (K - 1):, :]
    xi = conv * jax.nn.sigmoid(conv)

    xdbl = jnp.dot(xi.astype(jnp.bfloat16), wxp_ref[0],
                   preferred_element_type=jnp.float32)
    dtpre = jnp.dot(xdbl[:, 0:_LANES].astype(jnp.bfloat16), wdt_ref[0],
                    preferred_element_type=jnp.float32) + bdt_ref[0]
    dt = jnp.where(dtpre > 20.0, dtpre,
                   jnp.log1p(jnp.exp(jnp.minimum(dtpre, 20.0))))
    dt_s[0:CL, :] = dt
    u_s[0:CL, :] = dt * xi
    dt_s[CL:CL + 8, :] = jnp.zeros((8, DI), jnp.float32)
    u_s[CL:CL + 8, :] = jnp.zeros((8, DI), jnp.float32)
    bp_s[0:CL, :] = xdbl[:, _LANES:_LANES + N]
    cp_s[0:CL, :] = xdbl[:, 2 * _LANES:2 * _LANES + N]

    @pl.when(c == 0)
    def _zero_h():
        h_s[...] = jnp.zeros((N, DI), jnp.float32)

    eye = (jax.lax.broadcasted_iota(jnp.int32, (N, N), 0)
           == jax.lax.broadcasted_iota(jnp.int32, (N, N), 1))
    A2 = at_ref[0]

    def _col(ref, t):
        row = jnp.broadcast_to(ref[pl.ds(t, 1), :], (N, N))
        return jnp.sum(jnp.where(eye, row, 0.0), axis=1, keepdims=True)

    def body(t, h):
        dtr = dt_s[pl.ds(t, 1), :]
        dA = jnp.exp2(A2 * dtr)
        ur = u_s[pl.ds(t, 1), :]
        bcol = _col(bp_s, t)
        ccol = _col(cp_s, t)
        h = dA * h + ur * bcol
        ys_s[pl.ds(t, 1), :] = jnp.sum(h * ccol, axis=0, keepdims=True)
        return h

    UF = 4 if CL % 4 == 0 else (2 if CL % 2 == 0 else 1)

    def bodyu(i, h):
        for j in range(UF):
            h = body(UF * i + j, h)
        return h

    h = jax.lax.fori_loop(0, CL // UF, bodyu, h_s[...])
    h_s[...] = h

    y = ys_s[...] + xi * dsk_ref[0]
    y = y * (z * jax.nn.sigmoid(z))

    wrow = wT_ref[0, pl.ds(e, 1), :]
    wcol = jnp.swapaxes(wrow, 0, 1)
    contrib = jnp.dot((y * wcol).astype(jnp.bfloat16), wout_ref[0],
                      preferred_element_type=jnp.float32)

    sl = pl.ds(c * CL, CL)

    @pl.when(e == 0)
    def _first():
        out_ref[pl.ds(b, 1), sl, :] = contrib[None]

    @pl.when(e > 0)
    def _accum():
        out_ref[pl.ds(b, 1), sl, :] += contrib[None]


def kernel(x, W_gate, W_in, W_conv, b_conv, W_xproj, W_dt, b_dt, A_log,
           D_skip, W_out):
    B, L, D = x.shape
    E = W_gate.shape[0]
    DI, K = W_conv.shape[1], W_conv.shape[2]
    N = A_log.shape[2]
    DT_RANK = W_dt.shape[1]
    f32 = jnp.float32
    bf16 = jnp.bfloat16

    CL = 256 if L % 256 == 0 else (128 if L % 128 == 0 else L)
    NC = L // CL
    Epad = max(8, -(-E // 8) * 8)

    wg_pad = jnp.pad(W_gate.T, ((0, 0), (0, _LANES - E)))
    win_bf = W_in.astype(bf16)
    wct = jnp.pad(jnp.swapaxes(W_conv, 1, 2), ((0, 0), (0, 8 - K), (0, 0)))
    bconv3 = b_conv[:, None, :]
    wxp = jnp.concatenate([
        jnp.pad(W_xproj[:, :, :DT_RANK], ((0, 0), (0, 0), (0, _LANES - DT_RANK))),
        jnp.pad(W_xproj[:, :, DT_RANK:DT_RANK + N], ((0, 0), (0, 0), (0, _LANES - N))),
        jnp.pad(W_xproj[:, :, DT_RANK + N:], ((0, 0), (0, 0), (0, _LANES - N))),
    ], axis=-1).astype(bf16)
    wdt = jnp.pad(W_dt, ((0, 0), (0, _LANES - DT_RANK), (0, 0))).astype(bf16)
    bdt3 = b_dt[:, None, :]
    at = jnp.swapaxes(-jnp.exp(A_log), 1, 2) * 1.4426950408889634
    dsk3 = D_skip[:, None, :]
    wout_bf = W_out.astype(bf16)
    x_bf = x.astype(bf16)

    logitsT = pl.pallas_call(
        functools.partial(_logits_kernel, Epad, NC),
        grid=(B, NC),
        in_specs=[
            pl.BlockSpec((1, CL, D), lambda b, c: (b, c, 0)),
            pl.BlockSpec((D, _LANES), lambda b, c: (0, 0)),
        ],
        out_specs=pl.BlockSpec((Epad, CL), lambda b, c, NC=NC: (0, b * NC + c)),
        out_shape=jax.ShapeDtypeStruct((Epad, B * L), f32),
    )(x, wg_pad)
    wsc, dsc = _sc_router(logitsT, E)
    loadb, entb = pl.pallas_call(
        _stats_kernel,
        grid=(1,),
        in_specs=[
            pl.BlockSpec((Epad, B * L), lambda i: (0, 0)),
            pl.BlockSpec((8, B * L), lambda i: (0, 0)),
        ],
        out_specs=[
            pl.BlockSpec((8, _LANES), lambda i: (0, 0)),
            pl.BlockSpec((8, _LANES), lambda i: (0, 0)),
        ],
        out_shape=[
            jax.ShapeDtypeStruct((8, _LANES), f32),
            jax.ShapeDtypeStruct((8, _LANES), f32),
        ],
    )(wsc, dsc)
    wT = wsc.reshape(Epad, B, L).transpose(1, 0, 2)

    out = pl.pallas_call(
        functools.partial(_moe_kernel, DI, N, K, CL),
        grid=(E, B, NC),
        in_specs=[
            pl.BlockSpec((1, CL, D), lambda e, b, c: (b, c, 0)),
            pl.BlockSpec((1, D, 2 * DI), lambda e, b, c: (e, 0, 0)),
            pl.BlockSpec((1, 8, DI), lambda e, b, c: (e, 0, 0)),
            pl.BlockSpec((1, 1, DI), lambda e, b, c: (e, 0, 0)),
            pl.BlockSpec((1, DI, 3 * _LANES), lambda e, b, c: (e, 0, 0)),
            pl.BlockSpec((1, _LANES, DI), lambda e, b, c: (e, 0, 0)),
            pl.BlockSpec((1, 1, DI), lambda e, b, c: (e, 0, 0)),
            pl.BlockSpec((1, N, DI), lambda e, b, c: (e, 0, 0)),
            pl.BlockSpec((1, 1, DI), lambda e, b, c: (e, 0, 0)),
            pl.BlockSpec((1, DI, D), lambda e, b, c: (e, 0, 0)),
            pl.BlockSpec((1, Epad, CL), lambda e, b, c: (b, 0, c)),
        ],
        out_specs=pl.BlockSpec((B, L, D), lambda e, b, c: (0, 0, 0)),
        out_shape=jax.ShapeDtypeStruct((B, L, D), f32),
        scratch_shapes=[
            pltpu.VMEM((CL + 8, DI), f32),
            pltpu.VMEM((CL + 8, DI), f32),
            pltpu.VMEM((CL + 8, DI), f32),
            pltpu.VMEM((CL + 8, N), f32),
            pltpu.VMEM((CL + 8, N), f32),
            pltpu.VMEM((CL, DI), f32),
            pltpu.VMEM((N, DI), f32),
        ],
        compiler_params=pltpu.CompilerParams(
            dimension_semantics=("arbitrary",) * 3,
            vmem_limit_bytes=100 * 1024 * 1024,
        ),
    )(x_bf, win_bf, wct, bconv3, wxp, wdt, bdt3, at, dsk3, wout_bf, wT)

    load = loadb[:E, 0]
    routing_entropy = entb[0, 0] / (B * L)
    target = B * L * _TOPK / E
    ln = load / (target + 1e-8)
    mean = jnp.clip(jnp.mean(ln), 1e-8)
    std = jnp.std(ln, ddof=1)
    lb_loss = std / mean * 0.01 * E
    return out, lb_loss, load, routing_entropy
